# Initial kernel scaffold; baseline (speedup 1.0000x reference)
import jax
import jax.numpy as jnp
from jax.experimental import pallas as pl


def kernel(means, depths, gs_feats, intrinsics, extrinsics, W1, b1, W2, b2):
    raise NotImplementedError("write your pallas kernel here")


# speedup vs baseline: 1.7005x; 1.7005x over previous
import functools

import jax
import jax.numpy as jnp
import numpy as np
from jax import lax
from jax.experimental import pallas as pl
from jax.experimental.pallas import tpu as pltpu
from jax.experimental.pallas import tpu_sc as plsc

B, V, H, W, C = 2, 4, 256, 256, 32
GAMMA = 0.1
WIN = 1
HW = H * W

_PAIRS = [(i, j, k)
          for i in range(B)
          for j in range(V)
          for k in range(V)
          if j != k and (k - WIN) <= j <= (k + WIN)]
NP_ = len(_PAIRS)

_TAB0, _TAB1, _VALID1 = [], [], []
for i in range(B):
    for j in range(V):
        slots = [p for p, (pi, pj, pk) in enumerate(_PAIRS) if pi == i and pj == j]
        _TAB0.append(slots[0])
        _TAB1.append(slots[1] if len(slots) > 1 else 0)
        _VALID1.append(1.0 if len(slots) > 1 else 0.0)


def _proj_body(pts_ref, mm_ref, km_ref, off_ref, idx_ref, msk_ref, msum_ref):
    x = pts_ref[0, 0]
    y = pts_ref[0, 1]
    z3 = pts_ref[0, 2]
    eps = 1e-8

    def m(r, c):
        return mm_ref[0, r, c]

    pc0 = m(0, 0) * x + m(0, 1) * y + m(0, 2) * z3 + m(0, 3)
    pc1 = m(1, 0) * x + m(1, 1) * y + m(1, 2) * z3 + m(1, 3)
    pc2 = m(2, 0) * x + m(2, 1) * y + m(2, 2) * z3 + m(2, 3)

    valid_z = pc2 > eps
    inv_z = 1.0 / (pc2 + eps)
    xd = pc0 * inv_z
    yd = pc1 * inv_z

    def k(r, c):
        return km_ref[0, r, c]

    ndc0 = k(0, 0) * xd + k(0, 1) * yd + k(0, 2)
    ndc1 = k(1, 0) * xd + k(1, 1) * yd + k(1, 2)

    valid = ((ndc0 >= 0.0) & (ndc0 < 1.0) &
             (ndc1 >= 0.0) & (ndc1 < 1.0) & valid_z)
    xi = jnp.clip(jnp.floor(ndc0 * W).astype(jnp.int32), 0, W - 1)
    yi = jnp.clip(jnp.floor(ndc1 * H).astype(jnp.int32), 0, H - 1)

    mask_f = valid.astype(jnp.float32)
    idx_ref[0] = yi * W + xi + off_ref[0, 0, 0]
    msk_ref[0] = mask_f
    msum_ref[...] = jnp.sum(mask_f).reshape(1, 1, 1)


def _project_pairs(pts_h, mmat, kmat, offs):
    return pl.pallas_call(
        _proj_body,
        grid=(NP_,),
        in_specs=[
            pl.BlockSpec((1, 4, H, W), lambda p: (p, 0, 0, 0)),
            pl.BlockSpec((1, 3, 4), lambda p: (p, 0, 0)),
            pl.BlockSpec((1, 2, 3), lambda p: (p, 0, 0)),
            pl.BlockSpec((1, 1, 1), lambda p: (p, 0, 0)),
        ],
        out_specs=[
            pl.BlockSpec((1, H, W), lambda p: (p, 0, 0)),
            pl.BlockSpec((1, H, W), lambda p: (p, 0, 0)),
            pl.BlockSpec((1, 1, 1), lambda p: (p, 0, 0)),
        ],
        out_shape=[
            jax.ShapeDtypeStruct((NP_, H, W), jnp.int32),
            jax.ShapeDtypeStruct((NP_, H, W), jnp.float32),
            jax.ShapeDtypeStruct((NP_, 1, 1), jnp.float32),
        ],
    )(pts_h, mmat, kmat, offs)


_LANES_PAD = 128


def _sc_gather(table, idx):
    info = plsc.get_sparse_core_info()
    nw = info.num_cores * info.num_subcores
    nrows = idx.shape[0]
    per_w = nrows // nw
    chunk = 512
    nch = per_w // chunk
    mesh = plsc.VectorSubcoreMesh(core_axis_name="c", subcore_axis_name="s")

    @functools.partial(
        pl.kernel, mesh=mesh,
        out_type=jax.ShapeDtypeStruct((nrows, _LANES_PAD), jnp.float32),
        scratch_types=[
            pltpu.VMEM((chunk,), jnp.int32),
            pltpu.VMEM((chunk, _LANES_PAD), jnp.float32),
            pltpu.SemaphoreType.DMA,
        ],
    )
    def gather_k(table_hbm, idx_hbm, out_hbm, idx_v, rows_v, sem):
        wid = lax.axis_index("s") * info.num_cores + lax.axis_index("c")

        def body(it, carry):
            base = wid * per_w + it * chunk
            pltpu.sync_copy(idx_hbm.at[pl.ds(base, chunk)], idx_v)
            pltpu.async_copy(table_hbm.at[idx_v], rows_v, sem).wait()
            pltpu.sync_copy(rows_v, out_hbm.at[pl.ds(base, chunk)])
            return carry

        lax.fori_loop(0, nch, body, 0)

    return gather_k(table, idx)


_INV_HW_G = GAMMA / float(HW)
_RSQRT2 = 0.7071067811865476


_RS = 32
_NS = H // _RS


def _combine_body(tab0_ref, tab1_ref, gf_ref, g0_ref, g1_ref,
                  m0_ref, m1_ref, ms0_ref, ms1_ref, out_ref):
    del tab0_ref, tab1_ref
    f0 = ms0_ref[0, 0, 0] * _INV_HW_G
    f1 = ms1_ref[0, 0, 0] * _INV_HW_G
    xc = (gf_ref[0]
          + g0_ref[0] * (m0_ref[0] * f0)[:, :, None]
          + g1_ref[0] * (m1_ref[0] * f1)[:, :, None])
    out_ref[0] = xc * (1.0 / (1.0 + f0 + f1))


def _combine(gf, gath, msk, ms0, ms1):
    tab0 = jnp.asarray(_TAB0, dtype=jnp.int32)
    tab1 = jnp.asarray(_TAB1, dtype=jnp.int32)
    bv = B * V
    grid_spec = pltpu.PrefetchScalarGridSpec(
        num_scalar_prefetch=2,
        grid=(bv, _NS),
        in_specs=[
            pl.BlockSpec((1, _RS, W, C), lambda b, r, t0, t1: (b, r, 0, 0)),
            pl.BlockSpec((1, _RS, W, C), lambda b, r, t0, t1: (t0[b], r, 0, 0)),
            pl.BlockSpec((1, _RS, W, C), lambda b, r, t0, t1: (t1[b], r, 0, 0)),
            pl.BlockSpec((1, _RS, W), lambda b, r, t0, t1: (t0[b], r, 0)),
            pl.BlockSpec((1, _RS, W), lambda b, r, t0, t1: (t1[b], r, 0)),
            pl.BlockSpec((1, 1, 1), lambda b, r, t0, t1: (b, 0, 0)),
            pl.BlockSpec((1, 1, 1), lambda b, r, t0, t1: (b, 0, 0)),
        ],
        out_specs=pl.BlockSpec((1, _RS, W, C), lambda b, r, t0, t1: (b, r, 0, 0)),
    )
    return pl.pallas_call(
        _combine_body,
        grid_spec=grid_spec,
        out_shape=jax.ShapeDtypeStruct((bv, H, W, C), jnp.float32),
    )(tab0, tab1, gf, gath, gath, msk, msk, ms0, ms1)


def _conv_body(apply_gelu, x0_ref, x1_ref, x2_ref, w_ref, b_ref, out_ref):
    acc = jnp.zeros((_RS * W, C), jnp.float32) + b_ref[...]
    for dy, xr in enumerate((x0_ref, x1_ref, x2_ref)):
        for dx in range(3):
            xs = xr[0, :, pl.ds(dx, W), :].reshape(_RS * W, C)
            acc = acc + jnp.dot(xs, w_ref[dy, dx],
                                preferred_element_type=jnp.float32)
    if apply_gelu:
        acc = 0.5 * acc * (1.0 + lax.erf(acc * _RSQRT2))
    out_ref[0] = acc.reshape(_RS, W, C)


def _conv3x3(x, wt, bv_, apply_gelu):
    xpad = jnp.pad(x, ((0, 0), (1, 1), (1, 1), (0, 0)))
    shifts = [xpad[:, s:s + H] for s in range(3)]
    strip_spec = pl.BlockSpec((1, _RS, W + 2, C), lambda b, r: (b, r, 0, 0))
    return pl.pallas_call(
        functools.partial(_conv_body, apply_gelu),
        grid=(B * V, _NS),
        in_specs=[
            strip_spec, strip_spec, strip_spec,
            pl.BlockSpec((3, 3, C, C), lambda b, r: (0, 0, 0, 0)),
            pl.BlockSpec((1, C), lambda b, r: (0, 0)),
        ],
        out_specs=pl.BlockSpec((1, _RS, W, C), lambda b, r: (b, r, 0, 0)),
        out_shape=jax.ShapeDtypeStruct((B * V, H, W, C), jnp.float32),
    )(*shifts, wt, bv_)


def kernel(means, depths, gs_feats, intrinsics, extrinsics, W1, b1, W2, b2):
    del depths
    pi = np.array([p[0] for p in _PAIRS])
    pj = np.array([p[1] for p in _PAIRS])
    pk = np.array([p[2] for p in _PAIRS])

    w2c = jnp.linalg.inv(extrinsics)
    mmat = w2c[pi, pk, :3, :]
    kmat = intrinsics[pi, pk, :2, :]
    pts = means[pi, pj]
    pts_h = jnp.concatenate(
        [jnp.moveaxis(pts, -1, 1),
         jnp.ones((NP_, 1, H, W), jnp.float32)], axis=1)
    offs = ((pi * V + pk) * HW).astype(np.int32).reshape(NP_, 1, 1)
    offs = jnp.asarray(offs)

    idx, msk, msum = _project_pairs(pts_h, mmat, kmat, offs)

    table = jnp.pad(gs_feats.reshape(B * V * HW, C),
                    ((0, 0), (0, _LANES_PAD - C)))
    gath = _sc_gather(table, idx.reshape(NP_ * HW))[:, :C]
    gath = gath.reshape(NP_, H, W, C)

    ms = msum.reshape(NP_)
    ms0 = ms[jnp.asarray(_TAB0)].reshape(B * V, 1, 1)
    ms1 = (ms[jnp.asarray(_TAB1)]
           * jnp.asarray(_VALID1, jnp.float32)).reshape(B * V, 1, 1)

    w1t = jnp.transpose(W1, (2, 3, 1, 0))
    w2t = jnp.transpose(W2, (2, 3, 1, 0))
    gf = gs_feats.reshape(B * V, H, W, C)

    x = _combine(gf, gath, msk, ms0, ms1)
    x = _conv3x3(x, w1t, b1.reshape(1, C), apply_gelu=True)
    x = _conv3x3(x, w2t, b2.reshape(1, C), apply_gelu=False)
    return x.reshape(B, V, H, W, C)
